# baseline (speedup 1.0000x reference)
import jax
import jax.numpy as jnp
from jax import lax
from jax.experimental import pallas as pl
from jax.experimental.pallas import tpu as pltpu
from jax.experimental.pallas import tpu_sc as plsc

NC, NS, L = 2, 16, 16
NW = NC * NS
N, D = 10000, 128
E = 320000
EPAD = 327680
C = 128
EPT = EPAD // NW
CPT = EPT // C
G = C // L


def _zm_body(z_ref, d_ref, o_ref):
    o_ref[...] = z_ref[...] * d_ref[...]


def _compute_zm(z, d):
    return pl.pallas_call(
        _zm_body,
        grid=(10,),
        in_specs=[pl.BlockSpec((N // 10, D), lambda i: (i, 0))] * 2,
        out_specs=pl.BlockSpec((N // 10, D), lambda i: (i, 0)),
        out_shape=jax.ShapeDtypeStruct((N, D), jnp.float32),
    )(z, d)


def _sc_body(zm_hbm, sidx_hbm, didx_hbm, out_hbm,
             sidx_v, didx_v, srows0, drows0, srows1, drows1,
             out_v, sem0, sem1):
    wid = lax.axis_index("s") * NC + lax.axis_index("c")
    row0 = wid * CPT

    pltpu.sync_copy(sidx_hbm.at[pl.ds(row0, CPT)], sidx_v)
    pltpu.sync_copy(didx_hbm.at[pl.ds(row0, CPT)], didx_v)

    def issue(c, srows, drows, sem):
        pltpu.async_copy(zm_hbm.at[sidx_v.at[c]], srows, sem)
        pltpu.async_copy(zm_hbm.at[didx_v.at[c]], drows, sem)

    def wait(srows, drows, sem):
        pltpu.make_async_copy(zm_hbm.at[sidx_v.at[0]], srows, sem).wait()
        pltpu.make_async_copy(zm_hbm.at[didx_v.at[0]], drows, sem).wait()

    lane = lax.iota(jnp.int32, L)

    def compute(c, srows, drows):
        def group(g, carry):
            row = g * L + lane
            accs = [jnp.zeros((L,), jnp.float32) for _ in range(4)]
            for j in range(D):
                col = jnp.full((L,), j, jnp.int32)
                s = plsc.load_gather(srows, [row, col])
                d = plsc.load_gather(drows, [row, col])
                accs[j % 4] = accs[j % 4] + s * d
            tot = (accs[0] + accs[1]) + (accs[2] + accs[3])
            sig = 1.0 / (1.0 + jnp.exp(-tot))
            out_v[pl.ds(c * C + g * L, L)] = sig
            return carry
        lax.fori_loop(0, G, group, 0, unroll=False)

    issue(0, srows0, drows0, sem0)
    issue(1, srows1, drows1, sem1)

    def pair(k2, carry):
        k = 2 * k2
        wait(srows0, drows0, sem0)
        compute(k, srows0, drows0)

        @pl.when(k + 2 < CPT)
        def _():
            issue(k + 2, srows0, drows0, sem0)

        wait(srows1, drows1, sem1)
        compute(k + 1, srows1, drows1)

        @pl.when(k + 3 < CPT)
        def _():
            issue(k + 3, srows1, drows1, sem1)

        return carry

    lax.fori_loop(0, CPT // 2, pair, 0, unroll=False)
    pltpu.sync_copy(out_v, out_hbm.at[pl.ds(wid * EPT, EPT)])


_sc_call = pl.kernel(
    _sc_body,
    out_type=jax.ShapeDtypeStruct((EPAD,), jnp.float32),
    mesh=plsc.VectorSubcoreMesh(
        core_axis_name="c", subcore_axis_name="s",
        num_cores=NC, num_subcores=NS),
    compiler_params=pltpu.CompilerParams(needs_layout_passes=False),
    scratch_types=[
        pltpu.VMEM((CPT, C), jnp.int32),
        pltpu.VMEM((CPT, C), jnp.int32),
        pltpu.VMEM((C, D), jnp.float32),
        pltpu.VMEM((C, D), jnp.float32),
        pltpu.VMEM((C, D), jnp.float32),
        pltpu.VMEM((C, D), jnp.float32),
        pltpu.VMEM((EPT,), jnp.float32),
        pltpu.SemaphoreType.DMA,
        pltpu.SemaphoreType.DMA,
    ],
)


def kernel(z, edge_index, domain_embs):
    zm = _compute_zm(z, domain_embs)
    ei = edge_index.astype(jnp.int32)
    src = jnp.pad(ei[0], (0, EPAD - E)).reshape(EPAD // C, C)
    dst = jnp.pad(ei[1], (0, EPAD - E)).reshape(EPAD // C, C)
    out = _sc_call(zm, src, dst)
    return out[:E]


# speedup vs baseline: 2.7819x; 2.7819x over previous
import jax
import jax.numpy as jnp
from jax import lax
from jax.experimental import pallas as pl
from jax.experimental.pallas import tpu as pltpu
from jax.experimental.pallas import tpu_sc as plsc

NC, NS, L = 2, 16, 16
NW = NC * NS
N, D = 10000, 128
E = 320000
EPAD = 327680
C = 128
EPT = EPAD // NW
CPT = EPT // C
G = C // L


def _zm_body(z_ref, d_ref, o_ref):
    o_ref[...] = (z_ref[...] * d_ref[...]).astype(jnp.bfloat16)


def _compute_zm(z, d):
    return pl.pallas_call(
        _zm_body,
        grid=(10,),
        in_specs=[pl.BlockSpec((N // 10, D), lambda i: (i, 0))] * 2,
        out_specs=pl.BlockSpec((N // 10, D), lambda i: (i, 0)),
        out_shape=jax.ShapeDtypeStruct((N, D), jnp.bfloat16),
    )(z, d)


def _sc_body(zm_hbm, sidx_hbm, didx_hbm, out_hbm,
             sidx_v, didx_v, srows0, drows0, srows1, drows1,
             out_v, sem0, sem1):
    sid = lax.axis_index("s")
    wid = sid * NC + lax.axis_index("c")
    row0 = wid * CPT

    pltpu.sync_copy(sidx_hbm.at[pl.ds(row0, CPT)], sidx_v)
    pltpu.sync_copy(didx_hbm.at[pl.ds(row0, CPT)], didx_v)

    def issue(c, srows, drows, sem):
        pltpu.async_copy(zm_hbm.at[sidx_v.at[c]], srows, sem)
        pltpu.async_copy(zm_hbm.at[didx_v.at[c]], drows, sem)

    def wait(srows, drows, sem):
        pltpu.make_async_copy(zm_hbm.at[sidx_v.at[0]], srows, sem).wait()
        pltpu.make_async_copy(zm_hbm.at[didx_v.at[0]], drows, sem).wait()

    lane = lax.iota(jnp.int32, L)

    def compute(c, srows, drows):
        def group(g, carry):
            tot = jnp.zeros((L,), jnp.float32)
            for i in range(L):
                e = g * L + i
                acc = None
                for j in range(D // (2 * L)):
                    s2 = plsc.bitcast(srows[e, pl.ds(j * L, L)], jnp.bfloat16)
                    d2 = plsc.bitcast(drows[e, pl.ds(j * L, L)], jnp.bfloat16)
                    slo, shi = plsc.unpack(s2, format=plsc.PackFormat.INTERLEAVED)
                    dlo, dhi = plsc.unpack(d2, format=plsc.PackFormat.INTERLEAVED)
                    t = slo * dlo + shi * dhi
                    acc = t if acc is None else acc + t
                tot = jnp.where(lane == i, jnp.sum(acc), tot)
            sig = 1.0 / (1.0 + jnp.exp(-tot))
            out_v[pl.ds(c * C + g * L, L)] = sig
            return carry
        lax.fori_loop(0, G, group, 0, unroll=False)

    issue(0, srows0, drows0, sem0)
    issue(1, srows1, drows1, sem1)

    def pair(k2, carry):
        k = 2 * k2
        wait(srows0, drows0, sem0)
        compute(k, srows0, drows0)

        @pl.when(k + 2 < CPT)
        def _():
            issue(k + 2, srows0, drows0, sem0)

        wait(srows1, drows1, sem1)
        compute(k + 1, srows1, drows1)

        @pl.when(k + 3 < CPT)
        def _():
            issue(k + 3, srows1, drows1, sem1)

        return carry

    lax.fori_loop(0, CPT // 2, pair, 0, unroll=False)
    pltpu.sync_copy(out_v, out_hbm.at[pl.ds(wid * EPT, EPT)])


_sc_call = pl.kernel(
    _sc_body,
    out_type=jax.ShapeDtypeStruct((EPAD,), jnp.float32),
    mesh=plsc.VectorSubcoreMesh(
        core_axis_name="c", subcore_axis_name="s",
        num_cores=NC, num_subcores=NS),
    compiler_params=pltpu.CompilerParams(
        needs_layout_passes=False, use_tc_tiling_on_sc=False),
    scratch_types=[
        pltpu.VMEM((CPT, C), jnp.int32),
        pltpu.VMEM((CPT, C), jnp.int32),
        pltpu.VMEM((C, D // 2), jnp.int32),
        pltpu.VMEM((C, D // 2), jnp.int32),
        pltpu.VMEM((C, D // 2), jnp.int32),
        pltpu.VMEM((C, D // 2), jnp.int32),
        pltpu.VMEM((EPT,), jnp.float32),
        pltpu.SemaphoreType.DMA,
        pltpu.SemaphoreType.DMA,
    ],
)


def kernel(z, edge_index, domain_embs):
    zm_bf16 = _compute_zm(z, domain_embs)
    zm = jax.lax.bitcast_convert_type(
        zm_bf16.reshape(N, D // 2, 2), jnp.int32)
    ei = edge_index.astype(jnp.int32)
    src = jnp.pad(ei[0], (0, EPAD - E)).reshape(EPAD // C, C)
    dst = jnp.pad(ei[1], (0, EPAD - E)).reshape(EPAD // C, C)
    out = _sc_call(zm, src, dst)
    return out[:E]


# speedup vs baseline: 8.5934x; 3.0890x over previous
import jax
import jax.numpy as jnp
from jax import lax
from jax.experimental import pallas as pl
from jax.experimental.pallas import tpu as pltpu
from jax.experimental.pallas import tpu_sc as plsc

NC, NS, L = 2, 16, 16
NW = NC * NS
N, D = 10000, 128
E = 320000
EPAD = 327680
C = 80
EPT = EPAD // NW
CPT = EPT // C
G = C // L


def _zm_body(z_ref, d_ref, o_ref):
    o_ref[...] = (z_ref[...] * d_ref[...]).astype(jnp.bfloat16)


def _compute_zm(z, d):
    return pl.pallas_call(
        _zm_body,
        grid=(10,),
        in_specs=[pl.BlockSpec((N // 10, D), lambda i: (i, 0))] * 2,
        out_specs=pl.BlockSpec((N // 10, D), lambda i: (i, 0)),
        out_shape=jax.ShapeDtypeStruct((N, D), jnp.bfloat16),
    )(z, d)


def _sc_body(zm_hbm, sidx_hbm, didx_hbm, out_hbm,
             zm_sh, sidx_v, didx_v, srows0, drows0, srows1, drows1,
             out_v, sem0, sem1):
    sid = lax.axis_index("s")
    wid = sid * NC + lax.axis_index("c")
    row0 = wid * CPT

    def stage_piece(p, carry):
        piece = sid + NS * p

        @pl.when(piece < N // C)
        def _():
            r = piece * C
            pltpu.sync_copy(zm_hbm.at[pl.ds(r, C)], srows0)
            pltpu.sync_copy(srows0, zm_sh.at[pl.ds(r, C)])

        return carry

    lax.fori_loop(0, (N // C + NS - 1) // NS, stage_piece, 0)

    pltpu.sync_copy(sidx_hbm.at[pl.ds(row0, CPT)], sidx_v)
    pltpu.sync_copy(didx_hbm.at[pl.ds(row0, CPT)], didx_v)

    plsc.subcore_barrier()

    def issue(c, srows, drows, sem):
        pltpu.async_copy(zm_sh.at[sidx_v.at[c]], srows, sem)
        pltpu.async_copy(zm_sh.at[didx_v.at[c]], drows, sem)

    def wait(srows, drows, sem):
        pltpu.make_async_copy(zm_sh.at[sidx_v.at[0]], srows, sem).wait()
        pltpu.make_async_copy(zm_sh.at[didx_v.at[0]], drows, sem).wait()

    lane = lax.iota(jnp.int32, L)

    def compute(c, srows, drows):
        def group(g, carry):
            tot = jnp.zeros((L,), jnp.float32)
            for i in range(L):
                e = g * L + i
                acc = None
                for j in range(D // (2 * L)):
                    s2 = plsc.bitcast(srows[e, pl.ds(j * L, L)], jnp.bfloat16)
                    d2 = plsc.bitcast(drows[e, pl.ds(j * L, L)], jnp.bfloat16)
                    slo, shi = plsc.unpack(s2, format=plsc.PackFormat.INTERLEAVED)
                    dlo, dhi = plsc.unpack(d2, format=plsc.PackFormat.INTERLEAVED)
                    t = slo * dlo + shi * dhi
                    acc = t if acc is None else acc + t
                tot = jnp.where(lane == i, jnp.sum(acc), tot)
            sig = 1.0 / (1.0 + jnp.exp(-tot))
            out_v[pl.ds(c * C + g * L, L)] = sig
            return carry
        lax.fori_loop(0, G, group, 0, unroll=False)

    issue(0, srows0, drows0, sem0)
    issue(1, srows1, drows1, sem1)

    def pair(k2, carry):
        k = 2 * k2
        wait(srows0, drows0, sem0)
        compute(k, srows0, drows0)

        @pl.when(k + 2 < CPT)
        def _():
            issue(k + 2, srows0, drows0, sem0)

        wait(srows1, drows1, sem1)
        compute(k + 1, srows1, drows1)

        @pl.when(k + 3 < CPT)
        def _():
            issue(k + 3, srows1, drows1, sem1)

        return carry

    lax.fori_loop(0, CPT // 2, pair, 0, unroll=False)
    pltpu.sync_copy(out_v, out_hbm.at[pl.ds(wid * EPT, EPT)])


_sc_call = pl.kernel(
    _sc_body,
    out_type=jax.ShapeDtypeStruct((EPAD,), jnp.float32),
    mesh=plsc.VectorSubcoreMesh(
        core_axis_name="c", subcore_axis_name="s",
        num_cores=NC, num_subcores=NS),
    compiler_params=pltpu.CompilerParams(
        needs_layout_passes=False, use_tc_tiling_on_sc=False),
    scratch_types=[
        pltpu.VMEM_SHARED((N, D // 2), jnp.int32),
        pltpu.VMEM((CPT, C), jnp.int32),
        pltpu.VMEM((CPT, C), jnp.int32),
        pltpu.VMEM((C, D // 2), jnp.int32),
        pltpu.VMEM((C, D // 2), jnp.int32),
        pltpu.VMEM((C, D // 2), jnp.int32),
        pltpu.VMEM((C, D // 2), jnp.int32),
        pltpu.VMEM((EPT,), jnp.float32),
        pltpu.SemaphoreType.DMA,
        pltpu.SemaphoreType.DMA,
    ],
)


def kernel(z, edge_index, domain_embs):
    zm_bf16 = _compute_zm(z, domain_embs)
    zm = jax.lax.bitcast_convert_type(
        zm_bf16.reshape(N, D // 2, 2), jnp.int32)
    ei = edge_index.astype(jnp.int32)
    src = jnp.pad(ei[0], (0, EPAD - E)).reshape(EPAD // C, C)
    dst = jnp.pad(ei[1], (0, EPAD - E)).reshape(EPAD // C, C)
    out = _sc_call(zm, src, dst)
    return out[:E]
